# Initial kernel scaffold; baseline (speedup 1.0000x reference)
import jax
import jax.numpy as jnp
from jax.experimental import pallas as pl


def kernel(inputs, lookup_table):
    raise NotImplementedError("write your pallas kernel here")


# speedup vs baseline: 1.3620x; 1.3620x over previous
import functools

import jax
import jax.numpy as jnp
from jax import lax
from jax.experimental import pallas as pl
from jax.experimental.pallas import tpu as pltpu
from jax.experimental.pallas import tpu_sc as plsc

NUM_UNITS = 32
SCALE = float(NUM_UNITS) ** 0.5

_NC = 2
_NS = 16
_NW = _NC * _NS

_IDXW = 128
_K = 8
_CHUNK = _IDXW * _K


@functools.lru_cache(maxsize=None)
def _build(batch: int, vocab: int, units: int):
    assert batch % (_NW * _CHUNK) == 0
    bpw = batch // _NW
    nchunk = bpw // _CHUNK
    idx_rows_pw = bpw // _IDXW

    mesh = plsc.VectorSubcoreMesh(core_axis_name="c", subcore_axis_name="s")

    @functools.partial(
        pl.kernel,
        out_type=jax.ShapeDtypeStruct((batch, units), jnp.float32),
        mesh=mesh,
        scratch_types=[
            pltpu.VMEM((_K, _IDXW), jnp.int32),
            pltpu.VMEM((_CHUNK, units), jnp.float32),
            pltpu.SemaphoreType.DMA,
        ],
        compiler_params=pltpu.CompilerParams(use_tc_tiling_on_sc=False),
    )
    def emb(tbl_hbm, idx_hbm, out_hbm, idx_v, rows_v, sem):
        wid = lax.axis_index("s") * _NC + lax.axis_index("c")
        idx_row0 = wid * idx_rows_pw
        out_row0 = wid * bpw

        def chunk_body(g, carry):
            pltpu.sync_copy(
                idx_hbm.at[pl.ds(pl.multiple_of(idx_row0 + g * _K, 8), _K)],
                idx_v,
            )
            copies = [
                pltpu.make_async_copy(
                    tbl_hbm.at[idx_v.at[j]],
                    rows_v.at[pl.ds(j * _IDXW, _IDXW)],
                    sem,
                )
                for j in range(_K)
            ]
            for c in copies:
                c.start()
            for c in copies:
                c.wait()

            def scale_body(i, c2):
                rows_v[i, pl.ds(0, 16)] = rows_v[i, pl.ds(0, 16)] * SCALE
                rows_v[i, pl.ds(16, 16)] = rows_v[i, pl.ds(16, 16)] * SCALE
                return c2

            lax.fori_loop(0, _CHUNK, scale_body, 0)

            pltpu.sync_copy(
                rows_v,
                out_hbm.at[
                    pl.ds(pl.multiple_of(out_row0 + g * _CHUNK, 8), _CHUNK)
                ],
            )
            return carry

        lax.fori_loop(0, nchunk, chunk_body, 0)

    return emb


def kernel(inputs, lookup_table):
    b0, b1 = inputs.shape
    batch = b0 * b1
    vocab, units = lookup_table.shape
    idx = inputs.reshape(batch // _IDXW, _IDXW).astype(jnp.int32)
    out = _build(batch, vocab, units)(lookup_table, idx)
    return out.reshape(b0, b1, units)


# speedup vs baseline: 1.5552x; 1.1418x over previous
import functools

import jax
import jax.numpy as jnp
from jax import lax
from jax.experimental import pallas as pl
from jax.experimental.pallas import tpu as pltpu
from jax.experimental.pallas import tpu_sc as plsc

NUM_UNITS = 32
SCALE = float(NUM_UNITS) ** 0.5

_NC = 2
_NS = 16
_NW = _NC * _NS

_IDXW = 128
_K = 8
_CHUNK = _IDXW * _K
_NBUF = 3


@functools.lru_cache(maxsize=None)
def _build(batch: int, vocab: int, units: int):
    assert batch % (_NW * _CHUNK) == 0
    bpw = batch // _NW
    nchunk = bpw // _CHUNK
    idx_rows_pw = bpw // _IDXW

    mesh = plsc.VectorSubcoreMesh(core_axis_name="c", subcore_axis_name="s")

    @functools.partial(
        pl.kernel,
        out_type=jax.ShapeDtypeStruct((batch, units), jnp.float32),
        mesh=mesh,
        scratch_types=[
            pltpu.VMEM((idx_rows_pw, _IDXW), jnp.int32),
            pltpu.VMEM((_NBUF, _CHUNK, units), jnp.float32),
            [pltpu.SemaphoreType.DMA] * _NBUF,
            [pltpu.SemaphoreType.DMA] * _NBUF,
        ],
        compiler_params=pltpu.CompilerParams(use_tc_tiling_on_sc=False),
    )
    def emb(tbl_hbm, idx_hbm, out_hbm, idx_v, rows_v, gsems, osems):
        wid = lax.axis_index("s") * _NC + lax.axis_index("c")
        idx_row0 = pl.multiple_of(wid * idx_rows_pw, 8)
        out_row0 = pl.multiple_of(wid * bpw, 8)

        def gathers(g, b):
            return [
                pltpu.make_async_copy(
                    tbl_hbm.at[idx_v.at[g * _K + j]],
                    rows_v.at[b, pl.ds(j * _IDXW, _IDXW)],
                    gsems[b],
                )
                for j in range(_K)
            ]

        def write(g, b):
            return pltpu.make_async_copy(
                rows_v.at[b],
                out_hbm.at[pl.ds(out_row0 + g * _CHUNK, _CHUNK)],
                osems[b],
            )

        pltpu.sync_copy(idx_hbm.at[pl.ds(idx_row0, idx_rows_pw)], idx_v)
        for c in gathers(0, 0):
            c.start()
        for c in gathers(1, 1):
            c.start()

        for g in range(nchunk):
            b = g % _NBUF
            if g + 2 < nchunk:
                b2 = (g + 2) % _NBUF
                if g >= 1:
                    write(g - 1, b2).wait()
                for c in gathers(g + 2, b2):
                    c.start()
            for c in gathers(g, b):
                c.wait()

            @plsc.parallel_loop(0, _CHUNK, unroll=8)
            def _scale(i):
                rows_v[b, i, pl.ds(0, 16)] = rows_v[b, i, pl.ds(0, 16)] * SCALE
                rows_v[b, i, pl.ds(16, 16)] = rows_v[b, i, pl.ds(16, 16)] * SCALE

            write(g, b).start()

        for g in range(nchunk - _NBUF, nchunk):
            write(g, g % _NBUF).wait()

    return emb


def kernel(inputs, lookup_table):
    b0, b1 = inputs.shape
    batch = b0 * b1
    vocab, units = lookup_table.shape
    idx = inputs.reshape(batch // _IDXW, _IDXW).astype(jnp.int32)
    out = _build(batch, vocab, units)(lookup_table, idx)
    return out.reshape(b0, b1, units)
